# Initial kernel scaffold; baseline (speedup 1.0000x reference)
import jax
import jax.numpy as jnp
from jax.experimental import pallas as pl


def kernel(x, w1, g1, b1, w2, g2, b2):
    raise NotImplementedError("write your pallas kernel here")


# speedup vs baseline: 8.9568x; 8.9568x over previous
import jax
import jax.numpy as jnp
from jax.experimental import pallas as pl
from jax.experimental.pallas import tpu as pltpu

_EPS = 1e-5


def _choose_tile(hw):
    for t in (8192, 4096, 2048, 1024, 512, 256, 128):
        if hw % t == 0:
            return t
    return hw


def _moments(v):
    g = jax.lax.dot_general(v, v, (((1,), (1,)), ((), ())),
                            preferred_element_type=jnp.float32)
    s = jnp.sum(v, axis=1, keepdims=True)
    return jnp.concatenate([g, jnp.broadcast_to(s, g.shape)], axis=1)


def _stats1_body(x_ref, o_ref):
    o_ref[0, 0] = _moments(x_ref[0])


def _stats2_body(w1f_ref, s1_ref, x_ref, o_ref):
    h = jnp.dot(w1f_ref[...], x_ref[0],
                preferred_element_type=jnp.float32) + s1_ref[...]
    o_ref[0, 0] = _moments(jnp.maximum(h, 0.0))


def _apply_body(w1f_ref, s1_ref, w2f_ref, s2_ref, x_ref, o_ref):
    x = x_ref[0]
    r = jnp.maximum(jnp.dot(w1f_ref[...], x,
                            preferred_element_type=jnp.float32) + s1_ref[...],
                    0.0)
    z = jnp.dot(w2f_ref[...], r,
                preferred_element_type=jnp.float32) + s2_ref[...]
    o_ref[0] = (x * jax.nn.sigmoid(z)).astype(o_ref.dtype)


def kernel(x, w1, g1, b1, w2, g2, b2):
    N, C, H, W = x.shape
    K = w1.shape[0]
    HW = H * W
    M = N * HW
    T = _choose_tile(HW)
    nT = HW // T

    x3 = x.reshape(N, C, HW)
    grid = (N, nT)
    par = pltpu.CompilerParams(dimension_semantics=("parallel", "parallel"))

    x_spec = pl.BlockSpec((1, C, T), lambda n, t: (n, 0, t))
    w_spec = pl.BlockSpec((K, C), lambda n, t: (0, 0))
    sh_spec = pl.BlockSpec((K, 1), lambda n, t: (0, 0))

    ps1 = pl.pallas_call(
        _stats1_body,
        out_shape=jax.ShapeDtypeStruct((N, nT, C, 2 * C), jnp.float32),
        grid=grid,
        in_specs=[x_spec],
        out_specs=pl.BlockSpec((1, 1, C, 2 * C), lambda n, t: (n, t, 0, 0)),
        compiler_params=par,
    )(x3)
    ps1 = jnp.sum(ps1, axis=(0, 1))
    S, sx = ps1[:, :C], ps1[:, C]

    w1_f32 = w1.astype(jnp.float32)
    mean1 = (w1_f32 @ sx) / M
    sq1 = jnp.sum((w1_f32 @ S) * w1_f32, axis=1)
    var1 = jnp.maximum(sq1 / M - mean1 * mean1, 0.0)
    scale1 = g1.astype(jnp.float32) * jax.lax.rsqrt(var1 + _EPS)
    shift1 = b1.astype(jnp.float32) - mean1 * scale1
    w1f = scale1[:, None] * w1_f32

    ps2 = pl.pallas_call(
        _stats2_body,
        out_shape=jax.ShapeDtypeStruct((N, nT, K, 2 * K), jnp.float32),
        grid=grid,
        in_specs=[w_spec, sh_spec, x_spec],
        out_specs=pl.BlockSpec((1, 1, K, 2 * K), lambda n, t: (n, t, 0, 0)),
        compiler_params=par,
    )(w1f, shift1[:, None], x3)
    ps2 = jnp.sum(ps2, axis=(0, 1))
    R, sr = ps2[:, :K], ps2[:, K]

    w2_f32 = w2.astype(jnp.float32)
    mean2 = (w2_f32 @ sr) / M
    sq2 = jnp.sum((w2_f32 @ R) * w2_f32, axis=1)
    var2 = jnp.maximum(sq2 / M - mean2 * mean2, 0.0)
    scale2 = g2.astype(jnp.float32) * jax.lax.rsqrt(var2 + _EPS)
    shift2 = b2.astype(jnp.float32) - mean2 * scale2
    w2f = scale2[:, None] * w2_f32

    out3 = pl.pallas_call(
        _apply_body,
        out_shape=jax.ShapeDtypeStruct((N, C, HW), x.dtype),
        grid=grid,
        in_specs=[w_spec, sh_spec, w_spec, sh_spec, x_spec],
        out_specs=x_spec,
        compiler_params=par,
    )(w1f, shift1[:, None], w2f, shift2[:, None], x3)

    return out3.reshape(N, C, H, W)


# speedup vs baseline: 16.0833x; 1.7957x over previous
import jax
import jax.numpy as jnp
from jax.experimental import pallas as pl
from jax.experimental.pallas import tpu as pltpu

_EPS = 1e-5


def _moments(v):
    g = jax.lax.dot_general(v, v, (((1,), (1,)), ((), ())),
                            preferred_element_type=jnp.float32)
    s = jnp.sum(v, axis=1, keepdims=True)
    return jnp.concatenate([g, jnp.broadcast_to(s, g.shape)], axis=1)


def _make_stats1_body(B):
    def _body(x_ref, o_ref):
        acc = _moments(x_ref[0])
        for b in range(1, B):
            acc = acc + _moments(x_ref[b])
        o_ref[0] = acc
    return _body


def _make_stats2_body(B):
    def _body(w1f_ref, s1_ref, x_ref, o_ref):
        acc = None
        for b in range(B):
            h = jnp.dot(w1f_ref[...], x_ref[b],
                        preferred_element_type=jnp.float32) + s1_ref[...]
            m = _moments(jnp.maximum(h, 0.0))
            acc = m if acc is None else acc + m
        o_ref[0] = acc
    return _body


def _make_apply_body(B):
    def _body(w1f_ref, s1_ref, w2f_ref, s2_ref, x_ref, o_ref):
        for b in range(B):
            x = x_ref[b]
            r = jnp.maximum(jnp.dot(w1f_ref[...], x,
                                    preferred_element_type=jnp.float32)
                            + s1_ref[...], 0.0)
            z = jnp.dot(w2f_ref[...], r,
                        preferred_element_type=jnp.float32) + s2_ref[...]
            o_ref[b] = (x * jax.nn.sigmoid(z)).astype(o_ref.dtype)
    return _body


def kernel(x, w1, g1, b1, w2, g2, b2):
    N, C, H, W = x.shape
    K = w1.shape[0]
    HW = H * W
    M = N * HW

    B = 8 if N % 8 == 0 else 1
    G = N // B

    x3 = x.reshape(N, C, HW)
    par = pltpu.CompilerParams(dimension_semantics=("parallel",))

    x_spec = pl.BlockSpec((B, C, HW), lambda i: (i, 0, 0))
    w_spec = pl.BlockSpec((K, C), lambda i: (0, 0))
    sh_spec = pl.BlockSpec((K, 1), lambda i: (0, 0))

    ps1 = pl.pallas_call(
        _make_stats1_body(B),
        out_shape=jax.ShapeDtypeStruct((G, C, 2 * C), jnp.float32),
        grid=(G,),
        in_specs=[x_spec],
        out_specs=pl.BlockSpec((1, C, 2 * C), lambda i: (i, 0, 0)),
        compiler_params=par,
    )(x3)
    ps1 = jnp.sum(ps1, axis=0)
    S, sx = ps1[:, :C], ps1[:, C]

    w1_f32 = w1.astype(jnp.float32)
    mean1 = (w1_f32 @ sx) / M
    sq1 = jnp.sum((w1_f32 @ S) * w1_f32, axis=1)
    var1 = jnp.maximum(sq1 / M - mean1 * mean1, 0.0)
    scale1 = g1.astype(jnp.float32) * jax.lax.rsqrt(var1 + _EPS)
    shift1 = b1.astype(jnp.float32) - mean1 * scale1
    w1f = scale1[:, None] * w1_f32

    ps2 = pl.pallas_call(
        _make_stats2_body(B),
        out_shape=jax.ShapeDtypeStruct((G, K, 2 * K), jnp.float32),
        grid=(G,),
        in_specs=[w_spec, sh_spec, x_spec],
        out_specs=pl.BlockSpec((1, K, 2 * K), lambda i: (i, 0, 0)),
        compiler_params=par,
    )(w1f, shift1[:, None], x3)
    ps2 = jnp.sum(ps2, axis=0)
    R, sr = ps2[:, :K], ps2[:, K]

    w2_f32 = w2.astype(jnp.float32)
    mean2 = (w2_f32 @ sr) / M
    sq2 = jnp.sum((w2_f32 @ R) * w2_f32, axis=1)
    var2 = jnp.maximum(sq2 / M - mean2 * mean2, 0.0)
    scale2 = g2.astype(jnp.float32) * jax.lax.rsqrt(var2 + _EPS)
    shift2 = b2.astype(jnp.float32) - mean2 * scale2
    w2f = scale2[:, None] * w2_f32

    B3 = 4 if N % 4 == 0 else 1
    x3_spec = pl.BlockSpec((B3, C, HW), lambda i: (i, 0, 0))
    out3 = pl.pallas_call(
        _make_apply_body(B3),
        out_shape=jax.ShapeDtypeStruct((N, C, HW), x.dtype),
        grid=(N // B3,),
        in_specs=[w_spec, sh_spec, w_spec, sh_spec, x3_spec],
        out_specs=x3_spec,
        compiler_params=par,
    )(w1f, shift1[:, None], w2f, shift2[:, None], x3)

    return out3.reshape(N, C, H, W)


# speedup vs baseline: 30.8361x; 1.9173x over previous
import jax
import jax.numpy as jnp
from jax.experimental import pallas as pl
from jax.experimental.pallas import tpu as pltpu

_EPS = 1e-5


def _moments(v):
    g = jax.lax.dot_general(v, v, (((1,), (1,)), ((), ())),
                            preferred_element_type=jnp.float32)
    s = jnp.sum(v, axis=1, keepdims=True)
    return jnp.concatenate([g, jnp.broadcast_to(s, g.shape)], axis=1)


def _make_stats1_body(B, C, HW):
    def _body(x_ref, o_ref):
        acc = None
        for b in range(B):
            m = _moments(x_ref[b].reshape(C, HW))
            acc = m if acc is None else acc + m
        o_ref[0] = acc
    return _body


def _make_stats2_body(B, C, HW):
    def _body(w1f_ref, s1_ref, x_ref, o_ref):
        acc = None
        for b in range(B):
            h = jnp.dot(w1f_ref[...], x_ref[b].reshape(C, HW),
                        preferred_element_type=jnp.float32) + s1_ref[...]
            m = _moments(jnp.maximum(h, 0.0))
            acc = m if acc is None else acc + m
        o_ref[0] = acc
    return _body


def _make_apply_body(B, C, H, W):
    def _body(w1f_ref, s1_ref, w2f_ref, s2_ref, x_ref, o_ref):
        for b in range(B):
            x = x_ref[b].reshape(C, H * W)
            r = jnp.maximum(jnp.dot(w1f_ref[...], x,
                                    preferred_element_type=jnp.float32)
                            + s1_ref[...], 0.0)
            z = jnp.dot(w2f_ref[...], r,
                        preferred_element_type=jnp.float32) + s2_ref[...]
            out = (x * jax.nn.sigmoid(z)).astype(o_ref.dtype)
            o_ref[b] = out.reshape(C, H, W)
    return _body


def kernel(x, w1, g1, b1, w2, g2, b2):
    N, C, H, W = x.shape
    K = w1.shape[0]
    HW = H * W
    M = N * HW

    B = 8 if N % 8 == 0 else 1
    G = N // B

    par = pltpu.CompilerParams(dimension_semantics=("parallel",))

    x_spec = pl.BlockSpec((B, C, H, W), lambda i: (i, 0, 0, 0))
    w_spec = pl.BlockSpec((K, C), lambda i: (0, 0))
    sh_spec = pl.BlockSpec((K, 1), lambda i: (0, 0))

    ps1 = pl.pallas_call(
        _make_stats1_body(B, C, HW),
        out_shape=jax.ShapeDtypeStruct((G, C, 2 * C), jnp.float32),
        grid=(G,),
        in_specs=[x_spec],
        out_specs=pl.BlockSpec((1, C, 2 * C), lambda i: (i, 0, 0)),
        compiler_params=par,
    )(x)
    ps1 = jnp.sum(ps1, axis=0)
    S, sx = ps1[:, :C], ps1[:, C]

    w1_f32 = w1.astype(jnp.float32)
    mean1 = (w1_f32 @ sx) / M
    sq1 = jnp.sum((w1_f32 @ S) * w1_f32, axis=1)
    var1 = jnp.maximum(sq1 / M - mean1 * mean1, 0.0)
    scale1 = g1.astype(jnp.float32) * jax.lax.rsqrt(var1 + _EPS)
    shift1 = b1.astype(jnp.float32) - mean1 * scale1
    w1f = scale1[:, None] * w1_f32

    ps2 = pl.pallas_call(
        _make_stats2_body(B, C, HW),
        out_shape=jax.ShapeDtypeStruct((G, K, 2 * K), jnp.float32),
        grid=(G,),
        in_specs=[w_spec, sh_spec, x_spec],
        out_specs=pl.BlockSpec((1, K, 2 * K), lambda i: (i, 0, 0)),
        compiler_params=par,
    )(w1f, shift1[:, None], x)
    ps2 = jnp.sum(ps2, axis=0)
    R, sr = ps2[:, :K], ps2[:, K]

    w2_f32 = w2.astype(jnp.float32)
    mean2 = (w2_f32 @ sr) / M
    sq2 = jnp.sum((w2_f32 @ R) * w2_f32, axis=1)
    var2 = jnp.maximum(sq2 / M - mean2 * mean2, 0.0)
    scale2 = g2.astype(jnp.float32) * jax.lax.rsqrt(var2 + _EPS)
    shift2 = b2.astype(jnp.float32) - mean2 * scale2
    w2f = scale2[:, None] * w2_f32

    B3 = 4 if N % 4 == 0 else 1
    x3_spec = pl.BlockSpec((B3, C, H, W), lambda i: (i, 0, 0, 0))
    out = pl.pallas_call(
        _make_apply_body(B3, C, H, W),
        out_shape=jax.ShapeDtypeStruct((N, C, H, W), x.dtype),
        grid=(N // B3,),
        in_specs=[w_spec, sh_spec, w_spec, sh_spec, x3_spec],
        out_specs=x3_spec,
        compiler_params=par,
    )(w1f, shift1[:, None], w2f, shift2[:, None], x)

    return out


# speedup vs baseline: 31.1569x; 1.0104x over previous
import jax
import jax.numpy as jnp
from jax.experimental import pallas as pl
from jax.experimental.pallas import tpu as pltpu

_EPS = 1e-5


def _moments(v):
    g = jax.lax.dot_general(v, v, (((1,), (1,)), ((), ())),
                            preferred_element_type=jnp.float32)
    s = jnp.sum(v, axis=1, keepdims=True)
    return jnp.concatenate([g, jnp.broadcast_to(s, g.shape)], axis=1)


def _fold(ps, w, g, b, inv_m):
    C = w.shape[0]
    S, sv = ps[:, :C], ps[:, C:C + 1]
    mean = jnp.dot(w, sv, preferred_element_type=jnp.float32) * inv_m
    sq = jnp.sum(jnp.dot(w, S, preferred_element_type=jnp.float32) * w,
                 axis=1, keepdims=True)
    var = jnp.maximum(sq * inv_m - mean * mean, 0.0)
    scale = g * jax.lax.rsqrt(var + _EPS)
    return scale * w, b - mean * scale


def _make_stats1_body(B, C, HW):
    def _body(x_ref, o_ref):
        acc = None
        for b in range(B):
            m = _moments(x_ref[b].reshape(C, HW))
            acc = m if acc is None else acc + m
        o_ref[0] = acc
    return _body


def _make_stats2_body(B, C, HW, inv_m):
    def _body(w1_ref, g1_ref, b1_ref, ps1_ref, x_ref, o_ref):
        w1f, s1 = _fold(jnp.sum(ps1_ref[...], axis=0),
                        w1_ref[...], g1_ref[...], b1_ref[...], inv_m)
        acc = None
        for b in range(B):
            h = jnp.dot(w1f, x_ref[b].reshape(C, HW),
                        preferred_element_type=jnp.float32) + s1
            m = _moments(jnp.maximum(h, 0.0))
            acc = m if acc is None else acc + m
        o_ref[0] = acc
    return _body


def _make_apply_body(B, C, H, W, inv_m):
    def _body(w1_ref, g1_ref, b1_ref, w2_ref, g2_ref, b2_ref,
              ps1_ref, ps2_ref, x_ref, o_ref):
        w1f, s1 = _fold(jnp.sum(ps1_ref[...], axis=0),
                        w1_ref[...], g1_ref[...], b1_ref[...], inv_m)
        w2f, s2 = _fold(jnp.sum(ps2_ref[...], axis=0),
                        w2_ref[...], g2_ref[...], b2_ref[...], inv_m)
        for b in range(B):
            x = x_ref[b].reshape(C, H * W)
            r = jnp.maximum(jnp.dot(w1f, x,
                                    preferred_element_type=jnp.float32) + s1,
                            0.0)
            z = jnp.dot(w2f, r, preferred_element_type=jnp.float32) + s2
            out = (x * jax.nn.sigmoid(z)).astype(o_ref.dtype)
            o_ref[b] = out.reshape(C, H, W)
    return _body


def kernel(x, w1, g1, b1, w2, g2, b2):
    N, C, H, W = x.shape
    K = w1.shape[0]
    HW = H * W
    inv_m = 1.0 / (N * HW)

    B = 8 if N % 8 == 0 else 1
    G = N // B

    par = pltpu.CompilerParams(dimension_semantics=("parallel",))

    w1_f32 = w1.astype(jnp.float32)
    w2_f32 = w2.astype(jnp.float32)
    g1c, b1c = g1.astype(jnp.float32)[:, None], b1.astype(jnp.float32)[:, None]
    g2c, b2c = g2.astype(jnp.float32)[:, None], b2.astype(jnp.float32)[:, None]

    x_spec = pl.BlockSpec((B, C, H, W), lambda i: (i, 0, 0, 0))
    w_spec = pl.BlockSpec((K, C), lambda i: (0, 0))
    v_spec = pl.BlockSpec((K, 1), lambda i: (0, 0))
    ps_spec = pl.BlockSpec((G, K, 2 * K), lambda i: (0, 0, 0))

    ps1 = pl.pallas_call(
        _make_stats1_body(B, C, HW),
        out_shape=jax.ShapeDtypeStruct((G, C, 2 * C), jnp.float32),
        grid=(G,),
        in_specs=[x_spec],
        out_specs=pl.BlockSpec((1, C, 2 * C), lambda i: (i, 0, 0)),
        compiler_params=par,
    )(x)

    ps2 = pl.pallas_call(
        _make_stats2_body(B, C, HW, inv_m),
        out_shape=jax.ShapeDtypeStruct((G, K, 2 * K), jnp.float32),
        grid=(G,),
        in_specs=[w_spec, v_spec, v_spec, ps_spec, x_spec],
        out_specs=pl.BlockSpec((1, K, 2 * K), lambda i: (i, 0, 0)),
        compiler_params=par,
    )(w1_f32, g1c, b1c, ps1, x)

    out = pl.pallas_call(
        _make_apply_body(B, C, H, W, inv_m),
        out_shape=jax.ShapeDtypeStruct((N, C, H, W), x.dtype),
        grid=(G,),
        in_specs=[w_spec, v_spec, v_spec, w_spec, v_spec, v_spec,
                  ps_spec, ps_spec, x_spec],
        out_specs=x_spec,
        compiler_params=par,
    )(w1_f32, g1c, b1c, w2_f32, g2c, b2c, ps1, ps2, x)

    return out


# speedup vs baseline: 32.3555x; 1.0385x over previous
import jax
import jax.numpy as jnp
from jax.experimental import pallas as pl
from jax.experimental.pallas import tpu as pltpu

_EPS = 1e-5


def _moments(v):
    g = jax.lax.dot_general(v, v, (((1,), (1,)), ((), ())),
                            preferred_element_type=jnp.float32)
    s = jnp.sum(v, axis=1, keepdims=True)
    return jnp.concatenate([g, jnp.broadcast_to(s, g.shape)], axis=1)


def _fold(ps, w, g_row, b_row, inv_m):
    C = w.shape[0]
    g, b = jnp.transpose(g_row), jnp.transpose(b_row)
    S, sv = ps[:, :C], ps[:, C:C + 1]
    mean = jnp.dot(w, sv, preferred_element_type=jnp.float32) * inv_m
    sq = jnp.sum(jnp.dot(w, S, preferred_element_type=jnp.float32) * w,
                 axis=1, keepdims=True)
    var = jnp.maximum(sq * inv_m - mean * mean, 0.0)
    scale = g * jax.lax.rsqrt(var + _EPS)
    return scale * w, b - mean * scale


def _make_stats1_body(B, C, HW):
    def _body(x_ref, o_ref):
        acc = None
        for b in range(B):
            m = _moments(x_ref[b].reshape(C, HW))
            acc = m if acc is None else acc + m
        o_ref[0] = acc
    return _body


def _make_stats2_body(B, C, HW, inv_m):
    def _body(w1_ref, g1_ref, b1_ref, ps1_ref, x_ref, o_ref):
        w1f, s1 = _fold(jnp.sum(ps1_ref[...], axis=0),
                        w1_ref[...], g1_ref[...], b1_ref[...], inv_m)
        acc = None
        for b in range(B):
            h = jnp.dot(w1f, x_ref[b].reshape(C, HW),
                        preferred_element_type=jnp.float32) + s1
            m = _moments(jnp.maximum(h, 0.0))
            acc = m if acc is None else acc + m
        o_ref[0] = acc
    return _body


def _make_apply_body(B, C, H, W, inv_m):
    def _body(w1_ref, g1_ref, b1_ref, w2_ref, g2_ref, b2_ref,
              ps1_ref, ps2_ref, x_ref, o_ref):
        w1f, s1 = _fold(jnp.sum(ps1_ref[...], axis=0),
                        w1_ref[...], g1_ref[...], b1_ref[...], inv_m)
        w2f, s2 = _fold(jnp.sum(ps2_ref[...], axis=0),
                        w2_ref[...], g2_ref[...], b2_ref[...], inv_m)
        for b in range(B):
            x = x_ref[b].reshape(C, H * W)
            r = jnp.maximum(jnp.dot(w1f, x,
                                    preferred_element_type=jnp.float32) + s1,
                            0.0)
            z = jnp.dot(w2f, r, preferred_element_type=jnp.float32) + s2
            out = (x * jax.nn.sigmoid(z)).astype(o_ref.dtype)
            o_ref[b] = out.reshape(C, H, W)
    return _body


def kernel(x, w1, g1, b1, w2, g2, b2):
    N, C, H, W = x.shape
    K = w1.shape[0]
    HW = H * W
    inv_m = 1.0 / (N * HW)

    B = 8 if N % 8 == 0 else 1
    G = N // B

    par = pltpu.CompilerParams(dimension_semantics=("parallel",))

    w1_f32 = w1.astype(jnp.float32)
    w2_f32 = w2.astype(jnp.float32)
    g1c, b1c = g1.astype(jnp.float32)[None, :], b1.astype(jnp.float32)[None, :]
    g2c, b2c = g2.astype(jnp.float32)[None, :], b2.astype(jnp.float32)[None, :]

    x_spec = pl.BlockSpec((B, C, H, W), lambda i: (i, 0, 0, 0))
    w_spec = pl.BlockSpec((K, C), lambda i: (0, 0))
    v_spec = pl.BlockSpec((1, K), lambda i: (0, 0))
    ps_spec = pl.BlockSpec((G, K, 2 * K), lambda i: (0, 0, 0))

    ps1 = pl.pallas_call(
        _make_stats1_body(B, C, HW),
        out_shape=jax.ShapeDtypeStruct((G, C, 2 * C), jnp.float32),
        grid=(G,),
        in_specs=[x_spec],
        out_specs=pl.BlockSpec((1, C, 2 * C), lambda i: (i, 0, 0)),
        compiler_params=par,
    )(x)

    ps2 = pl.pallas_call(
        _make_stats2_body(B, C, HW, inv_m),
        out_shape=jax.ShapeDtypeStruct((G, K, 2 * K), jnp.float32),
        grid=(G,),
        in_specs=[w_spec, v_spec, v_spec, ps_spec, x_spec],
        out_specs=pl.BlockSpec((1, K, 2 * K), lambda i: (i, 0, 0)),
        compiler_params=par,
    )(w1_f32, g1c, b1c, ps1, x)

    out = pl.pallas_call(
        _make_apply_body(B, C, H, W, inv_m),
        out_shape=jax.ShapeDtypeStruct((N, C, H, W), x.dtype),
        grid=(G,),
        in_specs=[w_spec, v_spec, v_spec, w_spec, v_spec, v_spec,
                  ps_spec, ps_spec, x_spec],
        out_specs=x_spec,
        compiler_params=par,
    )(w1_f32, g1c, b1c, w2_f32, g2c, b2c, ps1, ps2, x)

    return out
